# baseline (speedup 1.0000x reference)
import jax
import jax.numpy as jnp
from jax.experimental import pallas as pl
from jax.experimental.pallas import tpu as pltpu

B = 16384
C = 102
R = 2048
N = B // R
H = 32


def _fusion_kernel(v_ref, t_ref, iw_ref, W1_ref, b1_ref, g1_ref, be1_ref,
                   W2_ref, b2_ref, W3_ref, b3_ref, g3_ref, be3_ref,
                   W4_ref, b4_ref, g4_ref, be4_ref, W5_ref, b5_ref, bc_ref,
                   out_ref, vs_ref, ts_ref, e_ref):
    i = pl.program_id(0)

    @pl.when(i < N)
    def _embed_phase():
        iw = iw_ref[...]
        cols = jax.lax.broadcasted_iota(jnp.int32, (R, C), 1)
        ones_col = jnp.full((C, 1), 1.0, jnp.float32)
        v = v_ref[...]
        t = t_ref[...]
        vs_ref[pl.ds(i * R, R), :] = v
        ts_ref[pl.ds(i * R, R), :] = t

        def feats_of(l):
            m = jnp.max(l, axis=1, keepdims=True)
            x = l - m
            ex = jnp.exp(x)
            z = jnp.dot(ex, ones_col, preferred_element_type=jnp.float32)
            s = jnp.dot(ex * x, ones_col,
                        preferred_element_type=jnp.float32)
            rz = 1.0 / z
            conf = rz
            ent = jnp.log(z) - s * rz
            feats = [ent, conf]
            pk = ex * rz
            for r in range(3):
                mk = jnp.max(pk, axis=1, keepdims=True)
                ik = jnp.min(jnp.where(pk == mk, cols, C), axis=1,
                             keepdims=True)
                sel = cols == ik
                feats.append(jnp.dot(jnp.where(sel, iw, 0.0), ones_col,
                                     preferred_element_type=jnp.float32))
                if r < 2:
                    pk = jnp.where(sel, -1.0, pk)
            return jnp.concatenate(feats, axis=1)

        e_ref[pl.ds(i * R, R), 0:5] = feats_of(v)
        e_ref[pl.ds(i * R, R), 5:10] = feats_of(t)

    @pl.when(i == N - 1)
    def _mlp_phase():
        rb = 1.0 / B

        def bn_relu(x, g, b):
            mu = jnp.sum(x, axis=0, keepdims=True) * rb
            m2 = jnp.sum(x * x, axis=0, keepdims=True) * rb
            a = g * jax.lax.rsqrt(m2 - mu * mu + 1e-5)
            return jnp.maximum(x * a + (b - mu * a), 0.0)

        two = lambda r: jnp.concatenate([r, r], axis=1)
        W1 = W1_ref[...]
        z5 = jnp.zeros((5, H), jnp.float32)
        Wbig = jnp.concatenate(
            [jnp.concatenate([W1, z5], axis=1),
             jnp.concatenate([z5, W1], axis=1)], axis=0)
        e = e_ref[:, 0:10]
        h1 = jnp.dot(e, Wbig, preferred_element_type=jnp.float32) \
            + two(b1_ref[...])
        x1 = bn_relu(h1, two(g1_ref[...]), two(be1_ref[...]))

        W3a = W3_ref[0:H, :]
        W3b = W3_ref[H:2 * H, :]
        Wc = jnp.concatenate(
            [jnp.dot(W2_ref[...], W3a, preferred_element_type=jnp.float32),
             jnp.dot(W2_ref[...], W3b, preferred_element_type=jnp.float32)],
            axis=0)
        bc3 = jnp.dot(b2_ref[...], W3a + W3b,
                      preferred_element_type=jnp.float32) + b3_ref[...]
        h3 = jnp.dot(x1, Wc, preferred_element_type=jnp.float32) + bc3
        x3 = bn_relu(h3, g3_ref[...], be3_ref[...])
        h4 = jnp.dot(x3, W4_ref[...], preferred_element_type=jnp.float32) \
            + b4_ref[...]
        x4 = bn_relu(h4, g4_ref[...], be4_ref[...])
        gate = jnp.dot(x4, W5_ref[...], preferred_element_type=jnp.float32) \
            + b5_ref[...]
        gm = jnp.max(gate, axis=1, keepdims=True)
        ge = jnp.exp(gate - gm)
        num = jnp.sum(ge * bc_ref[...], axis=1, keepdims=True)
        den = jnp.sum(ge, axis=1, keepdims=True)
        e_ref[:, 10:11] = num / den

    @pl.when(i >= N)
    def _fuse_phase():
        j = i - N
        w = e_ref[pl.ds(j * R, R), 10:11]
        out_ref[...] = (w * vs_ref[pl.ds(j * R, R), :]
                        + (2.0 - w) * ts_ref[pl.ds(j * R, R), :])


def kernel(v_logits, t_logits, init_weights, W1, b1, g1, be1, W2, b2,
           W3, b3, g3, be3, W4, b4, g4, be4, W5, b5, bin_center):
    row2d = lambda a: a.reshape(1, -1)
    logits_map = lambda i: (jnp.minimum(i, N - 1), 0)
    fixed = lambda shape: pl.BlockSpec(shape, lambda i: (0, 0))

    return pl.pallas_call(
        _fusion_kernel,
        grid=(2 * N,),
        in_specs=[
            pl.BlockSpec((R, C), logits_map),
            pl.BlockSpec((R, C), logits_map),
            fixed((1, C)),
            fixed((5, H)),
            fixed((1, H)),
            fixed((1, H)),
            fixed((1, H)),
            fixed((H, H)),
            fixed((1, H)),
            fixed((2 * H, H)),
            fixed((1, H)),
            fixed((1, H)),
            fixed((1, H)),
            fixed((H, H)),
            fixed((1, H)),
            fixed((1, H)),
            fixed((1, H)),
            fixed((H, 9)),
            fixed((1, 9)),
            fixed((1, 9)),
        ],
        out_specs=pl.BlockSpec((R, C),
                               lambda i: (jnp.where(i < N, 0, i - N), 0)),
        out_shape=jax.ShapeDtypeStruct((B, C), jnp.float32),
        scratch_shapes=[
            pltpu.VMEM((B, C), jnp.float32),
            pltpu.VMEM((B, C), jnp.float32),
            pltpu.VMEM((B, 16), jnp.float32),
        ],
    )(v_logits, t_logits, row2d(init_weights), W1, row2d(b1), row2d(g1),
      row2d(be1), W2, row2d(b2), W3, row2d(b3), row2d(g3), row2d(be3),
      W4, row2d(b4), row2d(g4), row2d(be4), W5, row2d(b5), row2d(bin_center))
